# Initial kernel scaffold; baseline (speedup 1.0000x reference)
import jax
import jax.numpy as jnp
from jax.experimental import pallas as pl


def kernel(x):
    raise NotImplementedError("write your pallas kernel here")


# speedup vs baseline: 13.0802x; 13.0802x over previous
import functools

import jax
import jax.numpy as jnp
import numpy as np
from jax import lax
from jax.experimental import pallas as pl
from jax.experimental.pallas import tpu as pltpu
from jax.experimental.pallas import tpu_sc as plsc

_SIZE = 20
_DIM = 2 ** _SIZE
_BATCH = 32
_D = 4096
_NROWS = _BATCH * (_DIM // _D)
_NW = 32
_ROWS_PER_W = _NROWS // _NW
_CH = 8
_NCH = _ROWS_PER_W // _CH

_jrow = np.arange(_NROWS, dtype=np.int32)
_SRC_ROWS = _jrow ^ ((_jrow & 0xAA) >> 1)

_mesh = plsc.VectorSubcoreMesh(core_axis_name="c", subcore_axis_name="s")


@functools.partial(
    pl.kernel,
    out_type=jax.ShapeDtypeStruct((_NROWS, _D), jnp.float32),
    mesh=_mesh,
    scratch_types=[
        pltpu.VMEM((_ROWS_PER_W,), jnp.int32),
        pltpu.VMEM((_CH, _D), jnp.float32),
        pltpu.VMEM((_CH, _D), jnp.float32),
        pltpu.SemaphoreType.DMA,
        pltpu.SemaphoreType.DMA,
        pltpu.SemaphoreType.DMA,
        pltpu.SemaphoreType.DMA,
    ],
)
def _cnot_perm_sc(x_hbm, idx_hbm, out_hbm, idx_v, buf_a, buf_b,
                  gsem_a, gsem_b, wsem_a, wsem_b):
    wid = lax.axis_index("s") * 2 + lax.axis_index("c")
    base = wid * _ROWS_PER_W
    pltpu.sync_copy(idx_hbm.at[pl.ds(base, _ROWS_PER_W)], idx_v)

    def gather(k, buf, sem):
        return pltpu.make_async_copy(
            x_hbm.at[idx_v.at[pl.ds(k * _CH, _CH)]], buf, sem)

    def write(k, buf, sem):
        return pltpu.make_async_copy(
            buf, out_hbm.at[pl.ds(base + k * _CH, _CH)], sem)

    gather(0, buf_a, gsem_a).start()

    @pl.loop(0, _NCH // 2)
    def _body(g):
        k_a = g * 2
        k_b = k_a + 1
        gather(k_a, buf_a, gsem_a).wait()
        gather(k_b, buf_b, gsem_b).start()
        write(k_a, buf_a, wsem_a).start()
        gather(k_b, buf_b, gsem_b).wait()
        write(k_a, buf_a, wsem_a).wait()
        gather((k_a + 2) & (_NCH - 1), buf_a, gsem_a).start()
        write(k_b, buf_b, wsem_b).start()
        write(k_b, buf_b, wsem_b).wait()

    gather(0, buf_a, gsem_a).wait()


def kernel(x):
    xr = x.reshape(_NROWS, _D)
    out = _cnot_perm_sc(xr, jnp.asarray(_SRC_ROWS))
    return out.reshape(_BATCH, _DIM)
